# baseline (speedup 1.0000x reference)
import jax
import jax.numpy as jnp
from jax.experimental import pallas as pl
from jax.experimental.pallas import tpu as pltpu
from jax.experimental.pallas import tpu_sc as plsc

_SIZE = 4096
_N = _SIZE * (_SIZE + 1) // 2

_NC = 2
_NS = 16
_NW = _NC * _NS
_RPW = _SIZE // _NW
_BUF = _SIZE + 8

_T = 1024
_NT = _SIZE // _T


_BLK = 256
_NBLK = _SIZE // _BLK


def _pipe(i_first, nrows, L, col0, v_hbm, u_hbm, in_a, in_b, out_a, out_b,
          sem_in, sem_out):
    bufl = L + 8

    def row_window(r):
        i = i_first + r
        start = i * _SIZE - (i * (i + 1)) // 2 + col0
        a = jnp.minimum((start // 8) * 8, _N - bufl)
        return i, start, a

    def issue_in(r, buf):
        _, _, a = row_window(r)
        pltpu.async_copy(v_hbm.at[pl.ds(a, bufl)], buf.at[pl.ds(0, bufl)], sem_in)

    def wait_in(buf):
        pltpu.make_async_copy(
            v_hbm.at[pl.ds(0, bufl)], buf.at[pl.ds(0, bufl)], sem_in
        ).wait()

    def shift_row(r, src, dst):
        _, start, a = row_window(r)
        sh = start - a

        def shift(k, carry2):
            dst[pl.ds(k * 16, 16)] = src[pl.ds(sh + k * 16, 16)]
            return carry2

        jax.lax.fori_loop(0, L // 16, shift, 0, unroll=8)

    def issue_out(r, buf):
        pltpu.async_copy(
            buf.at[pl.ds(0, L)], u_hbm.at[i_first + r, pl.ds(col0, L)], sem_out
        )

    def drain_out(buf):
        pltpu.make_async_copy(
            buf.at[pl.ds(0, L)], u_hbm.at[0, pl.ds(col0, L)], sem_out
        ).wait()

    issue_in(0, in_a)

    def step(t, carry):
        ra = 2 * t
        rb = 2 * t + 1
        issue_in(rb, in_b)
        wait_in(in_a)

        @pl.when(t > 0)
        def _():
            drain_out(out_a)
            drain_out(out_b)

        shift_row(ra, in_a, out_a)
        issue_out(ra, out_a)

        @pl.when(t < nrows // 2 - 1)
        def _():
            issue_in(rb + 1, in_a)

        wait_in(in_b)
        shift_row(rb, in_b, out_b)
        issue_out(rb, out_b)
        return carry

    jax.lax.fori_loop(0, nrows // 2, step, 0)
    drain_out(out_a)
    drain_out(out_b)


def _sc_gather_body(v_hbm, u_hbm, in_a, in_b, out_a, out_b, sem_in, sem_out):
    c = jax.lax.axis_index("c")
    s = jax.lax.axis_index("s")
    wid = s * _NC + c
    p = wid // 4
    q = wid % 4
    nrows = _BLK // 4

    for pp in range(_NBLK // 2):
        @pl.when(p == pp)
        def _(pp=pp):
            for b in (pp, _NBLK - 1 - pp):
                col0 = _BLK * b
                _pipe(_BLK * b + q * nrows, nrows, _SIZE - col0, col0,
                      v_hbm, u_hbm, in_a, in_b, out_a, out_b, sem_in, sem_out)


_NPAIR = _NT * (_NT + 1) // 2
_LOS = tuple(lo for lo in range(_NT) for hi in range(lo, _NT))
_HIS = tuple(hi for lo in range(_NT) for hi in range(lo, _NT))


def _sym_kernel(los_ref, his_ref, u_ref, out_ref):
    s = pl.program_id(0)
    lo = los_ref[s]
    hi = his_ref[s]
    u = u_ref[...]

    @pl.when(lo == hi)
    def _():
        rows = jax.lax.broadcasted_iota(jnp.int32, (_T, _T), 0)
        cols = jax.lax.broadcasted_iota(jnp.int32, (_T, _T), 1)
        um = jnp.where(cols >= rows, u, 0.0)
        out_ref[...] = um + um.T - jnp.where(rows == cols, u, 0.0)

    @pl.when(lo < hi)
    def _():
        out_ref[...] = u.T


def kernel(upper_tri_vector):
    sc_gather = pl.kernel(
        _sc_gather_body,
        out_type=jax.ShapeDtypeStruct((_SIZE, _SIZE), jnp.float32),
        mesh=plsc.VectorSubcoreMesh(
            core_axis_name="c", subcore_axis_name="s", num_cores=_NC,
            num_subcores=_NS,
        ),
        scratch_types=[
            pltpu.VMEM((_BUF,), jnp.float32),
            pltpu.VMEM((_BUF,), jnp.float32),
            pltpu.VMEM((_SIZE,), jnp.float32),
            pltpu.VMEM((_SIZE,), jnp.float32),
            pltpu.SemaphoreType.DMA,
            pltpu.SemaphoreType.DMA,
        ],
    )
    u = sc_gather(upper_tri_vector)

    out = pl.pallas_call(
        _sym_kernel,
        grid_spec=pltpu.PrefetchScalarGridSpec(
            num_scalar_prefetch=2,
            grid=(_NPAIR,),
            in_specs=[
                pl.BlockSpec((_T, _T), lambda s, los, his: (los[s], his[s]))
            ],
            out_specs=pl.BlockSpec((_T, _T), lambda s, los, his: (his[s], los[s])),
        ),
        out_shape=jax.ShapeDtypeStruct((_SIZE, _SIZE), jnp.float32),
        input_output_aliases={2: 0},
    )(jnp.asarray(_LOS, jnp.int32), jnp.asarray(_HIS, jnp.int32), u)
    return out


# speedup vs baseline: 1.0868x; 1.0868x over previous
import jax
import jax.numpy as jnp
from jax.experimental import pallas as pl
from jax.experimental.pallas import tpu as pltpu
from jax.experimental.pallas import tpu_sc as plsc

_SIZE = 4096
_N = _SIZE * (_SIZE + 1) // 2

_NC = 2
_NS = 16
_NW = _NC * _NS
_RPW = _SIZE // _NW
_BUF = _SIZE + 8

_T = 1024
_NT = _SIZE // _T


_BLK = 256
_NBLK = _SIZE // _BLK


def _pipe(i_first, nrows, L, col0, v_hbm, u_hbm, ins, outs,
          sem_in, sem_out):
    bufl = L + 8
    nd = 3

    def row_window(r):
        i = i_first + r
        start = i * _SIZE - (i * (i + 1)) // 2 + col0
        a = jnp.minimum((start // 8) * 8, _N - bufl)
        return i, start, a

    def issue_in(r, buf):
        _, _, a = row_window(r)
        pltpu.async_copy(v_hbm.at[pl.ds(a, bufl)], buf.at[pl.ds(0, bufl)], sem_in)

    def wait_in(buf):
        pltpu.make_async_copy(
            v_hbm.at[pl.ds(0, bufl)], buf.at[pl.ds(0, bufl)], sem_in
        ).wait()

    def shift_row(r, src, dst):
        _, start, a = row_window(r)
        sh = start - a

        def shift(k, carry2):
            dst[pl.ds(k * 16, 16)] = src[pl.ds(sh + k * 16, 16)]
            return carry2

        jax.lax.fori_loop(0, L // 16, shift, 0, unroll=8)

    def issue_out(r, buf):
        pltpu.async_copy(
            buf.at[pl.ds(0, L)], u_hbm.at[i_first + r, pl.ds(col0, L)], sem_out
        )

    def drain_out(buf):
        pltpu.make_async_copy(
            buf.at[pl.ds(0, L)], u_hbm.at[0, pl.ds(col0, L)], sem_out
        ).wait()

    for j in range(nd):
        issue_in(j, ins[j])

    def step(t, carry):
        for j in range(nd):
            r = nd * t + j
            wait_in(ins[j])

            @pl.when(t > 0)
            def _():
                drain_out(outs[j])

            shift_row(r, ins[j], outs[j])
            issue_out(r, outs[j])

            @pl.when(r + nd < nrows)
            def _():
                issue_in(r + nd, ins[j])
        return carry

    jax.lax.fori_loop(0, nrows // nd, step, 0)

    r_last = (nrows // nd) * nd
    wait_in(ins[0])
    drain_out(outs[0])
    shift_row(r_last, ins[0], outs[0])
    issue_out(r_last, outs[0])
    drain_out(outs[1])
    drain_out(outs[2])
    drain_out(outs[0])


def _sc_gather_body(v_hbm, u_hbm, i0, i1, i2, o0, o1, o2, sem_in, sem_out):
    ins = [i0, i1, i2]
    outs = [o0, o1, o2]
    c = jax.lax.axis_index("c")
    s = jax.lax.axis_index("s")
    wid = s * _NC + c
    p = wid // 4
    q = wid % 4
    nrows = _BLK // 4

    for pp in range(_NBLK // 2):
        @pl.when(p == pp)
        def _(pp=pp):
            for b in (pp, _NBLK - 1 - pp):
                col0 = _BLK * b
                _pipe(_BLK * b + q * nrows, nrows, _SIZE - col0, col0,
                      v_hbm, u_hbm, ins, outs, sem_in, sem_out)


_NPAIR = _NT * (_NT + 1) // 2
_LOS = tuple(lo for lo in range(_NT) for hi in range(lo, _NT))
_HIS = tuple(hi for lo in range(_NT) for hi in range(lo, _NT))


def _sym_kernel(los_ref, his_ref, u_ref, out_ref):
    s = pl.program_id(0)
    lo = los_ref[s]
    hi = his_ref[s]
    u = u_ref[...]

    @pl.when(lo == hi)
    def _():
        rows = jax.lax.broadcasted_iota(jnp.int32, (_T, _T), 0)
        cols = jax.lax.broadcasted_iota(jnp.int32, (_T, _T), 1)
        um = jnp.where(cols >= rows, u, 0.0)
        out_ref[...] = um + um.T - jnp.where(rows == cols, u, 0.0)

    @pl.when(lo < hi)
    def _():
        out_ref[...] = u.T


def kernel(upper_tri_vector):
    sc_gather = pl.kernel(
        _sc_gather_body,
        out_type=jax.ShapeDtypeStruct((_SIZE, _SIZE), jnp.float32),
        mesh=plsc.VectorSubcoreMesh(
            core_axis_name="c", subcore_axis_name="s", num_cores=_NC,
            num_subcores=_NS,
        ),
        scratch_types=[
            pltpu.VMEM((_BUF,), jnp.float32),
            pltpu.VMEM((_BUF,), jnp.float32),
            pltpu.VMEM((_BUF,), jnp.float32),
            pltpu.VMEM((_SIZE,), jnp.float32),
            pltpu.VMEM((_SIZE,), jnp.float32),
            pltpu.VMEM((_SIZE,), jnp.float32),
            pltpu.SemaphoreType.DMA,
            pltpu.SemaphoreType.DMA,
        ],
    )
    u = sc_gather(upper_tri_vector)

    out = pl.pallas_call(
        _sym_kernel,
        grid_spec=pltpu.PrefetchScalarGridSpec(
            num_scalar_prefetch=2,
            grid=(_NPAIR,),
            in_specs=[
                pl.BlockSpec((_T, _T), lambda s, los, his: (los[s], his[s]))
            ],
            out_specs=pl.BlockSpec((_T, _T), lambda s, los, his: (his[s], los[s])),
        ),
        out_shape=jax.ShapeDtypeStruct((_SIZE, _SIZE), jnp.float32),
        input_output_aliases={2: 0},
    )(jnp.asarray(_LOS, jnp.int32), jnp.asarray(_HIS, jnp.int32), u)
    return out
